# Initial kernel scaffold; baseline (speedup 1.0000x reference)
import jax
import jax.numpy as jnp
from jax.experimental import pallas as pl


def kernel(input_, weight):
    raise NotImplementedError("write your pallas kernel here")


# speedup vs baseline: 1.5311x; 1.5311x over previous
import functools

import numpy as np
import jax
import jax.numpy as jnp
from jax import lax
from jax.experimental import pallas as pl
from jax.experimental.pallas import tpu as pltpu
from jax.experimental.pallas import tpu_sc as plsc

PAD = 0
D = 32
L = 16
IB = 128


def _lane_min(v):
    dnums = lax.GatherDimensionNumbers(
        offset_dims=(), collapsed_slice_dims=(0,), start_index_map=(0,)
    )
    for sh in (8, 4, 2, 1):
        perm = (lax.iota(jnp.int32, L) + sh) % L
        rot = lax.gather(
            v, perm[:, None], dnums, (1,),
            mode=lax.GatherScatterMode.PROMISE_IN_BOUNDS,
        )
        v = jnp.minimum(v, rot)
    return v[0]


@functools.partial(jax.jit, static_argnums=(2, 3))
def _emb_lookup(weight, idx, B, C):
    info = plsc.get_sparse_core_info()
    NC, NS = info.num_cores, info.num_subcores
    NW = NC * NS
    b_per_w = B // NW
    n_chunks = b_per_w // C
    gathers_per_chunk = C // IB
    mesh = plsc.VectorSubcoreMesh(core_axis_name="c", subcore_axis_name="s")

    @functools.partial(
        pl.kernel,
        mesh=mesh,
        out_type=jax.ShapeDtypeStruct((B, D), jnp.float32),
        compiler_params=pltpu.CompilerParams(
            needs_layout_passes=False, use_tc_tiling_on_sc=False
        ),
        scratch_types=[
            pltpu.VMEM((C,), jnp.int32),
            pltpu.VMEM((C, D), jnp.float32),
            pltpu.SemaphoreType.DMA,
        ],
    )
    def k(weight_hbm, idx_hbm, out_hbm, idx_v, rows_v, sem):
        wid = lax.axis_index("s") * NC + lax.axis_index("c")
        base = wid * b_per_w
        zeros = jnp.zeros((L,), jnp.float32)

        def chunk_body(g, _):
            off = pl.multiple_of(base + g * C, C)
            pltpu.sync_copy(idx_hbm.at[pl.ds(off, C)], idx_v)
            for j in range(gathers_per_chunk):
                pltpu.async_copy(
                    weight_hbm.at[idx_v.at[pl.ds(j * IB, IB)]],
                    rows_v.at[pl.ds(j * IB, IB)],
                    sem,
                )

            def min_body(i, acc):
                return jnp.minimum(acc, idx_v[pl.ds(i * L, L)])

            acc = lax.fori_loop(
                0, C // L, min_body, jnp.full((L,), 2**30, jnp.int32),
                unroll=False,
            )
            min_idx = _lane_min(acc)

            for j in range(gathers_per_chunk):
                pltpu.make_async_copy(
                    weight_hbm.at[idx_v.at[pl.ds(j * IB, IB)]],
                    rows_v.at[pl.ds(j * IB, IB)],
                    sem,
                ).wait()

            @pl.when(min_idx == PAD)
            def _():
                def fix_body(i, _):
                    iv = idx_v[pl.ds(i * L, L)]
                    m = iv == PAD
                    rowpos = lax.iota(jnp.int32, L) + i * L
                    for j in range(D):
                        plsc.store_scatter(
                            rows_v,
                            [rowpos, jnp.full((L,), j, jnp.int32)],
                            zeros,
                            mask=m,
                        )
                    return 0

                lax.fori_loop(0, C // L, fix_body, 0, unroll=False)

            pltpu.sync_copy(rows_v, out_hbm.at[pl.ds(off, C)])
            return 0

        lax.fori_loop(0, n_chunks, chunk_body, 0, unroll=False)

    return k(weight, idx)


def kernel(input_, weight):
    shape = input_.shape
    B = input_.size
    idx = input_.reshape(B)
    out = _emb_lookup(weight, idx, B, 1024)
    return out.reshape(*shape, D)
